# Initial kernel scaffold; baseline (speedup 1.0000x reference)
import jax
import jax.numpy as jnp
from jax.experimental import pallas as pl


def kernel(x, wih_t, whh_t, b_lstm, w1_t, b1, w2_t, b2):
    raise NotImplementedError("write your pallas kernel here")


# speedup vs baseline: 1.0854x; 1.0854x over previous
import jax
import jax.numpy as jnp
from jax import lax
from jax.experimental import pallas as pl
from jax.experimental.pallas import tpu as pltpu


def _round_up(n, m):
    return ((n + m - 1) // m) * m


def _lstm_kernel(x_ref,
                 wih_ref,
                 whh_ref,
                 b_ref,
                 w1_ref,
                 b1_ref,
                 w2_ref,
                 b2_ref,
                 out_ref,
                 xw_ref):
    T, BB, D = x_ref.shape
    H = whh_ref.shape[0]
    G = 4 * H

    whh = whh_ref[...]

    x2d = x_ref[...].reshape(T * BB, D)
    bias = jnp.broadcast_to(b_ref[...], (T * BB, G))
    xw_ref[...] = jnp.dot(x2d, wih_ref[...],
                          preferred_element_type=jnp.float32) + bias

    lane = lax.broadcasted_iota(jnp.int32, (1, G), 1)
    g_lane = (lane >= 2 * H) & (lane < 3 * H)
    coef = jnp.where(g_lane, 1.0, 0.5).astype(jnp.float32)
    off = jnp.where(g_lane, 0.0, 0.5).astype(jnp.float32)

    h_bf = jnp.zeros((BB, H), jnp.bfloat16)
    c = jnp.zeros((BB, H), jnp.float32)

    for t in range(T):
        gates = (xw_ref[t * BB:(t + 1) * BB, :]
                 + jnp.dot(h_bf, whh, preferred_element_type=jnp.float32))
        th = jnp.tanh(gates * coef)
        act = th * coef + off
        i_g = act[:, 0 * H:1 * H]
        f_g = act[:, 1 * H:2 * H]
        g_g = act[:, 2 * H:3 * H]
        o_g = act[:, 3 * H:4 * H]
        c = f_g * c + i_g * g_g
        h = o_g * jnp.tanh(c)
        h_bf = h.astype(jnp.bfloat16)

    z1 = jnp.dot(h, w1_ref[...], preferred_element_type=jnp.float32) + b1_ref[...]
    z1 = jnp.maximum(z1, 0.0)
    z2 = jnp.dot(z1, w2_ref[...], preferred_element_type=jnp.float32) + b2_ref[...]
    out_ref[...] = jax.nn.sigmoid(z2)


def kernel(x, wih_t, whh_t, b_lstm, w1_t, b1, w2_t, b2):
    B, T, D = x.shape
    H = whh_t.shape[0]
    G = 4 * H
    F1 = w1_t.shape[1]
    O = w2_t.shape[1]

    batch_block = min(128, _round_up(B, 8))
    batch_block = max(8, _round_up(batch_block, 8))
    B_pad = _round_up(B, batch_block)
    OP = _round_up(O, 128)
    if B_pad != B:
        x = jnp.pad(x, ((0, B_pad - B), (0, 0), (0, 0)))
    w2p = jnp.pad(w2_t, ((0, 0), (0, OP - O)))
    b2p = jnp.pad(b2, ((0, 0), (0, OP - O)))

    x_tm = jnp.transpose(x, (1, 0, 2)).astype(jnp.bfloat16)
    nb = B_pad // batch_block

    out = pl.pallas_call(
        _lstm_kernel,
        out_shape=jax.ShapeDtypeStruct((B_pad, OP), jnp.float32),
        grid_spec=pltpu.PrefetchScalarGridSpec(
            num_scalar_prefetch=0,
            grid=(nb,),
            in_specs=[
                pl.BlockSpec((T, batch_block, D), lambda i: (0, i, 0)),
                pl.BlockSpec((D, G), lambda i: (0, 0)),
                pl.BlockSpec((H, G), lambda i: (0, 0)),
                pl.BlockSpec((1, G), lambda i: (0, 0)),
                pl.BlockSpec((H, F1), lambda i: (0, 0)),
                pl.BlockSpec((1, F1), lambda i: (0, 0)),
                pl.BlockSpec((F1, OP), lambda i: (0, 0)),
                pl.BlockSpec((1, OP), lambda i: (0, 0)),
            ],
            out_specs=pl.BlockSpec((batch_block, OP), lambda i: (i, 0)),
            scratch_shapes=[pltpu.VMEM((T * batch_block, G), jnp.float32)],
        ),
        compiler_params=pltpu.CompilerParams(
            dimension_semantics=("parallel",),
            vmem_limit_bytes=100 * 1024 * 1024,
        ),
    )(x_tm,
      wih_t.astype(jnp.bfloat16), whh_t.astype(jnp.bfloat16), b_lstm,
      w1_t, b1, w2p, b2p)

    return out[:B, :O]
